# baseline (speedup 1.0000x reference)
import jax
import jax.numpy as jnp
from jax.experimental import pallas as pl
from jax.experimental.pallas import tpu as pltpu

_B, _C, _T, _TD = 8, 256, 512, 16
_K = 3


def _relu(a):
    return jnp.maximum(a, 0.0)


def _dot(a, b):
    return jax.lax.dot_general(a, b, (((1,), (0,)), ((), ())),
                               preferred_element_type=jnp.float32)


def _shift_right(a):
    return jnp.concatenate([jnp.zeros_like(a[:, :1]), a[:, :-1]], axis=1)


def _shift_left(a):
    return jnp.concatenate([a[:, 1:], jnp.zeros_like(a[:, :1])], axis=1)


def _bd_expand(w, groups, ig, ntaps=1, tap=0):
    o = w.shape[0]
    og = o // groups
    n = groups * ig
    csub = jax.lax.broadcasted_iota(jnp.int32, (w.shape[1], n), 0)
    clane = jax.lax.broadcasted_iota(jnp.int32, (w.shape[1], n), 1)
    pick = (csub == ntaps * (clane % ig) + tap).astype(jnp.float32)
    tiled = _dot(w, pick)
    sub = jax.lax.broadcasted_iota(jnp.int32, (o, n), 0)
    lane = jax.lax.broadcasted_iota(jnp.int32, (o, n), 1)
    return jnp.where((lane // ig) == (sub // og), tiled, 0.0)


def _layer(x, tf_col, valid, p):
    t1 = _relu(_dot(p['t1'], x) + p['bt1'])
    t2 = _relu(_shift_right(_dot(p['t2s'][0], t1)) + _dot(p['t2s'][1], t1)
               + _shift_left(_dot(p['t2s'][2], t1)) + p['bt2'])
    tout = _relu(_dot(p['t3'], t2) + p['bt3'])

    g = jax.lax.dot_general(x, x, (((0,), (0,)), ((), ())),
                            preferred_element_type=jnp.float32)
    sq = jnp.sum(x * x, axis=0, keepdims=True)
    sq_col = jnp.swapaxes(sq, 0, 1)
    score = jnp.where(valid, sq_col - 2.0 * g, 1e9)
    sub = jax.lax.broadcasted_iota(jnp.int32, (_T, _T), 0)

    uv = _dot(p['uv'], x)
    u = uv[:128] + p['sb1']
    v = uv[128:]
    ve = _dot(p['uv'][128:], tf_col)

    s1 = [_relu(u + ve)]
    for j in range(_K):
        mn = jnp.min(score, axis=0, keepdims=True)
        idx = jnp.min(jnp.where(score == mn, sub, _T),
                      axis=0, keepdims=True)
        sel = sub == idx
        if j < _K - 1:
            score = jnp.where(sel, 1e9, score)
        oh = sel.astype(jnp.float32)
        nbr = jax.lax.dot_general(v, oh, (((1,), (0,)), ((), ())),
                                  preferred_element_type=jnp.float32)
        s1.append(_relu(u + nbr))
    s1 = jnp.concatenate(s1, axis=1)
    s2 = _relu(_dot(p['s2'], s1) + p['sb2'])
    s3 = _dot(p['s3'], s2) + p['sb3']
    m = jnp.maximum(jnp.maximum(s3[:, :_T], s3[:, _T:2 * _T]),
                    jnp.maximum(s3[:, 2 * _T:3 * _T], s3[:, 3 * _T:]))
    sout = _relu(m)
    return _relu(tout + x + sout)


def _body(x_ref, seg_ref, topic_ref, wbb_ref, bbb_ref, wbt_ref, bbt_ref,
          g1t1, g1bt1, g1t2, g1bt2, g1t3, g1bt3, g1s1, g1sb1, g1s2, g1sb2,
          g1s3, g1sb3,
          g2t1, g2bt1, g2t2, g2bt2, g2t3, g2bt3, g2s1, g2sb1, g2s2, g2sb2,
          g2s3, g2sb3,
          out_ref,
          bb_s, wbt_s, t2a_s, s2a_s, uva_s, t2b_s, s2b_s, uvb_s):
    b = pl.program_id(0)

    @pl.when(b == 0)
    def _prep():
        for j in range(3):
            bb_s[j] = _bd_expand(wbb_ref[...], 4, 64, 3, j)
            t2a_s[j] = _bd_expand(g1t2[...], 32, 4, 3, j)
            t2b_s[j] = _bd_expand(g2t2[...], 32, 4, 3, j)
        wbt_s[...] = _bd_expand(wbt_ref[...], 4, 4)
        s2a_s[...] = _bd_expand(g1s2[...], 32, 4)
        s2b_s[...] = _bd_expand(g2s2[...], 32, 4)
        w1a = g1s1[...]
        uva_s[...] = jnp.concatenate(
            [w1a[:, :_C] - w1a[:, _C:], w1a[:, _C:]], axis=0)
        w1b = g2s1[...]
        uvb_s[...] = jnp.concatenate(
            [w1b[:, :_C] - w1b[:, _C:], w1b[:, _C:]], axis=0)

    x0 = x_ref[0]
    seg = jnp.maximum(seg_ref[b], _K + 1)
    valid = jax.lax.broadcasted_iota(jnp.int32, (_T, 1), 0) < seg

    base = _relu(_shift_right(_dot(bb_s[0], x0))
                 + _dot(bb_s[1], x0)
                 + _shift_left(_dot(bb_s[2], x0)) + bbb_ref[...])

    trow = topic_ref[0]
    tf_col = _relu(jnp.sum(wbt_s[...] * trow, axis=1, keepdims=True)
                   + bbt_ref[...])

    p1 = dict(t1=g1t1[...], bt1=g1bt1[...], t2s=t2a_s, bt2=g1bt2[...],
              t3=g1t3[...], bt3=g1bt3[...], uv=uva_s[...], sb1=g1sb1[...],
              s2=s2a_s[...], sb2=g1sb2[...], s3=g1s3[...], sb3=g1sb3[...])
    p2 = dict(t1=g2t1[...], bt1=g2bt1[...], t2s=t2b_s, bt2=g2bt2[...],
              t3=g2t3[...], bt3=g2bt3[...], uv=uvb_s[...], sb1=g2sb1[...],
              s2=s2b_s[...], sb2=g2sb2[...], s3=g2s3[...], sb3=g2sb3[...])

    x1 = _layer(base, tf_col, valid, p1)
    out_ref[0] = _layer(x1, tf_col, valid, p2)


def _prep_gcn(g):
    col = lambda b: b[:, None]
    return [
        g['tw1'][:, :, 0], col(g['tb1']),
        g['tw2'].reshape(128, 12), col(g['tb2']),
        g['tw3'][:, :, 0], col(g['tb3']),
        g['sw1'][:, :, 0, 0], col(g['sb1']),
        g['sw2'][:, :, 0, 0], col(g['sb2']),
        g['sw3'][:, :, 0, 0], col(g['sb3']),
    ]


def kernel(snip_feature, seg_lens, topic_embedding, w_bb, b_bb, w_bt, b_bt,
           g1, g2, interpret=False):
    inputs = ([snip_feature, seg_lens.astype(jnp.int32),
               topic_embedding[:, None, :],
               w_bb.reshape(_C, 192), b_bb[:, None],
               w_bt[:, :, 0], b_bt[:, None]]
              + _prep_gcn(g1) + _prep_gcn(g2))

    rep = lambda a: pl.BlockSpec(a.shape, lambda b: (0,) * a.ndim)
    in_specs = [pl.BlockSpec((1, _C, _T), lambda b: (b, 0, 0)),
                pl.BlockSpec(memory_space=pltpu.SMEM),
                pl.BlockSpec((1, 1, _TD), lambda b: (b, 0, 0))]
    in_specs += [rep(a) for a in inputs[3:]]

    scratch = [pltpu.VMEM((3, _C, _C), jnp.float32),
               pltpu.VMEM((_C, _TD), jnp.float32),
               pltpu.VMEM((3, 128, 128), jnp.float32),
               pltpu.VMEM((128, 128), jnp.float32),
               pltpu.VMEM((_C, _C), jnp.float32),
               pltpu.VMEM((3, 128, 128), jnp.float32),
               pltpu.VMEM((128, 128), jnp.float32),
               pltpu.VMEM((_C, _C), jnp.float32)]

    return pl.pallas_call(
        _body,
        grid=(_B,),
        in_specs=in_specs,
        out_specs=pl.BlockSpec((1, _C, _T), lambda b: (b, 0, 0)),
        out_shape=jax.ShapeDtypeStruct((_B, _C, _T), jnp.float32),
        scratch_shapes=scratch,
        interpret=interpret,
    )(*inputs)
